# baseline (speedup 1.0000x reference)
import functools

import numpy as np
import jax
import jax.numpy as jnp
from jax import lax
from jax.experimental import pallas as pl
from jax.experimental.pallas import tpu as pltpu
from jax.experimental.pallas import tpu_sc as plsc

MAXD = 7
H = 128
ND = [4 ** d for d in range(MAXD + 1)]
OFF = np.concatenate([[0], np.cumsum(ND)]).astype(np.int32)
TOT = int(OFF[-1])
NWORK = 32
FEAT_CHUNK = 688
TOT_PAD = FEAT_CHUNK * NWORK
TAPS = [(dy, dx) for dy in (-1, 0, 1) for dx in (-1, 0, 1)]


def _interleave(x):
    x = np.asarray(x, np.int64) & 0xFFFF
    x = (x | (x << 8)) & 0x00FF00FF
    x = (x | (x << 4)) & 0x0F0F0F0F
    x = (x | (x << 2)) & 0x33333333
    x = (x | (x << 1)) & 0x55555555
    return x


def _tables(d):
    n = 1 << d
    N = n * n
    i = np.arange(N)
    x, y = i % n, i // n
    morton_of_grid = (_interleave(x) | (_interleave(y) << 1)).astype(np.int32)
    grid_of_morton = np.empty(N, np.int32)
    grid_of_morton[morton_of_grid] = i.astype(np.int32)
    xc = (x.astype(np.float32) + 0.5) / np.float32(n)
    yc = (y.astype(np.float32) + 0.5) / np.float32(n)
    dn = np.full(N, np.float32(d) / np.float32(MAXD), np.float32)
    pos = np.stack([xc, yc, dn], 1)
    freqs = (2.0 ** np.arange(6)).astype(np.float32).reshape(1, 1, -1)
    xx = pos[..., None].astype(np.float32) * np.float32(np.pi) * 2.0 * freqs
    enc = np.concatenate([np.sin(xx), np.cos(xx)], -1).reshape(N, -1)
    posf = np.concatenate([pos, enc], 1).astype(np.float32)
    masks = np.stack([(x > 0).astype(np.float32),
                      (x < n - 1).astype(np.float32)], 1)
    return morton_of_grid, grid_of_morton, np.concatenate([posf, masks], 1)


_TABS = [_tables(d) for d in range(MAXD + 1)]

_FEAT_SRC = np.zeros(TOT_PAD, np.int32)
for _d in range(MAXD + 1):
    _FEAT_SRC[OFF[_d]:OFF[_d + 1]] = OFF[_d] + _TABS[_d][0]

_SX = {}
for _d in range(1, MAXD + 1):
    _n = 1 << _d
    _m = _n // 2
    _rows = max(_m, 8)
    _s = np.zeros((_rows, _n), np.float32)
    for _X in range(_m):
        _s[_X, 2 * _X] = 0.25
        _s[_X, 2 * _X + 1] = 0.25
    _SX[_d] = _s


@functools.lru_cache(maxsize=None)
def _sc_mesh():
    return plsc.VectorSubcoreMesh(core_axis_name="c", subcore_axis_name="s",
                                  num_cores=2, num_subcores=16)


def _sc_feat_body(src_hbm, idx_hbm, out_hbm, tab_v, idx_v, out_v):
    w = lax.axis_index("s") * 2 + lax.axis_index("c")
    base = w * FEAT_CHUNK
    pltpu.sync_copy(src_hbm, tab_v)
    pltpu.sync_copy(idx_hbm.at[pl.ds(base, FEAT_CHUNK)], idx_v)

    def body(i, c):
        ids = idx_v[pl.ds(i * 16, 16)]
        out_v[pl.ds(i * 16, 16)] = plsc.load_gather(tab_v, [ids])
        return c

    lax.fori_loop(0, FEAT_CHUNK // 16, body, 0)
    pltpu.sync_copy(out_v, out_hbm.at[pl.ds(base, FEAT_CHUNK)])


@functools.lru_cache(maxsize=None)
def _sc_feat_gather():
    return pl.kernel(
        _sc_feat_body,
        out_type=jax.ShapeDtypeStruct((TOT_PAD,), jnp.float32),
        mesh=_sc_mesh(),
        compiler_params=pltpu.CompilerParams(needs_layout_passes=False),
        scratch_types=[
            pltpu.VMEM((TOT_PAD,), jnp.float32),
            pltpu.VMEM((FEAT_CHUNK,), jnp.int32),
            pltpu.VMEM((FEAT_CHUNK,), jnp.float32),
        ],
    )

_SC2_DEPTHS = list(range(2, MAXD + 1))


def _sc_unshuffle_body(*refs):
    nd = len(_SC2_DEPTHS)
    e_refs = refs[0:nd]
    i_refs = refs[nd:2 * nd]
    o_refs = refs[2 * nd:3 * nd]
    idx_v, rows_v, sem = refs[3 * nd:]
    w = lax.axis_index("s") * 2 + lax.axis_index("c")

    for t, d in enumerate(_SC2_DEPTHS):
        N = ND[d]
        rw = N if N < 8 * NWORK else N // NWORK
        chunk = min(rw, 128)
        nch = rw // chunk

        def run(base, t=t, rw=rw, chunk=chunk, nch=nch):
            pltpu.sync_copy(i_refs[t].at[pl.ds(base, rw)], idx_v.at[pl.ds(0, rw)])
            for k in range(nch):
                pltpu.async_copy(
                    e_refs[t].at[idx_v.at[pl.ds(k * chunk, chunk)]],
                    rows_v.at[pl.ds(0, chunk)], sem).wait()
                pltpu.sync_copy(rows_v.at[pl.ds(0, chunk)],
                                o_refs[t].at[pl.ds(base + k * chunk, chunk)])

        if N < 8 * NWORK:
            @pl.when(w == 0)
            def _():
                run(0)
        else:
            run(w * rw)


@functools.lru_cache(maxsize=None)
def _sc_unshuffle():
    return pl.kernel(
        _sc_unshuffle_body,
        out_type=tuple(jax.ShapeDtypeStruct((ND[d], H), jnp.float32)
                       for d in _SC2_DEPTHS),
        mesh=_sc_mesh(),
        scratch_types=[
            pltpu.VMEM((512,), jnp.int32),
            pltpu.VMEM((128, H), jnp.float32),
            pltpu.SemaphoreType.DMA,
        ],
    )


def _shift_zero(T, s, N):
    if s == 0:
        return T
    z = jnp.zeros((abs(s), T.shape[1]), T.dtype)
    if s > 0:
        return jnp.concatenate([T[s:], z], axis=0)
    return jnp.concatenate([z, T[:s]], axis=0)


def _tc_body(*refs):
    it = iter(refs)
    X = [next(it) for _ in range(8)]
    inW = next(it)
    inb = next(it)
    Wcat = {d: next(it) for d in range(1, 7)}
    convb = {d: next(it) for d in range(1, 7)}
    embW = [next(it) for _ in range(8)]
    embb = [next(it) for _ in range(8)]
    g2 = [next(it) for _ in range(8)]
    b2 = [next(it) for _ in range(8)]
    Sx = {d: next(it) for d in range(1, 8)}
    E = [next(it) for _ in range(8)]

    Wv = inW[...]
    bv = inb[...]
    for d in range(8):
        A = X[d][...][:, 0:40].astype(jnp.bfloat16)
        E[d][...] = jnp.dot(A, Wv, preferred_element_type=jnp.float32) + bv

    for d in range(7, 0, -1):
        n = 1 << d
        m = n // 2
        Sxv = Sx[d][...]
        for Y in range(m):
            rA = E[d][pl.ds((2 * Y) * n, n), :]
            rB = E[d][pl.ds((2 * Y + 1) * n, n), :]
            ch = jnp.dot(Sxv, (rA + rB).astype(jnp.bfloat16),
                         preferred_element_type=jnp.float32)
            E[d - 1][pl.ds(Y * m, m), :] = E[d - 1][pl.ds(Y * m, m), :] + ch[:m]
        dc = d - 1
        if dc >= 1:
            nc = 1 << dc
            Nc = ND[dc]
            hv = E[dc][...].astype(jnp.bfloat16)
            Wc = Wcat[dc][...]
            Xm = X[dc][:, 40:42]
            sums = []
            for dx in (-1, 0, 1):
                t = None
                for dy in (-1, 0, 1):
                    j = (dy + 1) * 3 + (dx + 1)
                    T = jnp.dot(hv, Wc[:, j * H:(j + 1) * H],
                                preferred_element_type=jnp.float32)
                    T = _shift_zero(T, dy * nc + dx, Nc)
                    t = T if t is None else t + T
                sums.append(t)
            acc = (convb[dc][...] + sums[1]
                   + Xm[:, 0:1] * sums[0] + Xm[:, 1:2] * sums[2])
            E[dc][...] = jnp.maximum(acc, 0.0)

    for d in range(8):
        hv = E[d][...].astype(jnp.bfloat16)
        z = jnp.dot(hv, embW[d][...], preferred_element_type=jnp.float32) + embb[d][...]
        mu = jnp.mean(z, axis=1, keepdims=True)
        zc = z - mu
        var = jnp.mean(zc * zc, axis=1, keepdims=True)
        zn = zc * lax.rsqrt(var + 1e-5)
        E[d][...] = zn * g2[d][...] + b2[d][...]


_tc_pyramid = pl.pallas_call(
    _tc_body,
    out_shape=tuple(jax.ShapeDtypeStruct((ND[d], H), jnp.float32)
                    for d in range(8)),
)


def kernel(features_0, features_1, features_2, features_3, features_4,
           features_5, features_6, features_7, in_proj_W, in_proj_b,
           conv_W, conv_b, emb_W, emb_b, ln_g, ln_b, depth_gain):
    feats = [features_0, features_1, features_2, features_3, features_4,
             features_5, features_6, features_7]
    f32 = jnp.float32

    feats_all = jnp.concatenate(
        [f.reshape(-1) for f in feats]
        + [jnp.zeros((TOT_PAD - TOT,), f32)])
    fg = _sc_feat_gather()(feats_all, jnp.asarray(_FEAT_SRC))

    ops = []
    for d in range(8):
        fcol = fg[OFF[d]:OFF[d] + ND[d]].reshape(ND[d], 1)
        ops.append(jnp.concatenate([fcol, jnp.asarray(_TABS[d][2])], axis=1))
    bf16 = jnp.bfloat16
    ops.append(in_proj_W.astype(bf16))
    ops.append(in_proj_b.reshape(1, H))
    for d in range(1, 7):
        ops.append(conv_W[d].reshape(9, H, H).transpose(1, 0, 2)
                   .reshape(H, 9 * H).astype(bf16))
    for d in range(1, 7):
        ops.append(conv_b[d].reshape(1, H))
    for d in range(8):
        ops.append(emb_W[d].astype(bf16))
    for d in range(8):
        ops.append(emb_b[d].reshape(1, H))
    for d in range(8):
        ops.append((depth_gain[d] * ln_g[d]).reshape(1, H))
    for d in range(8):
        ops.append((depth_gain[d] * ln_b[d]).reshape(1, H))
    for d in range(1, 8):
        ops.append(jnp.asarray(_SX[d]).astype(bf16))

    Eg = _tc_pyramid(*ops)

    Em = _sc_unshuffle()(
        *[Eg[d] for d in _SC2_DEPTHS],
        *[jnp.asarray(_TABS[d][1]) for d in _SC2_DEPTHS])

    return (Eg[0], Eg[1]) + tuple(Em)


# speedup vs baseline: 1.3335x; 1.3335x over previous
import functools

import numpy as np
import jax
import jax.numpy as jnp
from jax import lax
from jax.experimental import pallas as pl
from jax.experimental.pallas import tpu as pltpu
from jax.experimental.pallas import tpu_sc as plsc

MAXD = 7
H = 128
ND = [4 ** d for d in range(MAXD + 1)]
NWORK = 32
TAPS = [(dy, dx) for dy in (-1, 0, 1) for dx in (-1, 0, 1)]
BF16 = jnp.bfloat16


def _interleave(x):
    x = np.asarray(x, np.int64) & 0xFFFF
    x = (x | (x << 8)) & 0x00FF00FF
    x = (x | (x << 4)) & 0x0F0F0F0F
    x = (x | (x << 2)) & 0x33333333
    x = (x | (x << 1)) & 0x55555555
    return x


def _tables(d):
    n = 1 << d
    N = n * n
    i = np.arange(N)
    x, y = i % n, i // n
    morton_of_grid = (_interleave(x) | (_interleave(y) << 1)).astype(np.int32)
    grid_of_morton = np.empty(N, np.int32)
    grid_of_morton[morton_of_grid] = i.astype(np.int32)
    xc = (x.astype(np.float32) + 0.5) / np.float32(n)
    yc = (y.astype(np.float32) + 0.5) / np.float32(n)
    dn = np.full(N, np.float32(d) / np.float32(MAXD), np.float32)
    pos = np.stack([xc, yc, dn], 1)
    freqs = (2.0 ** np.arange(6)).astype(np.float32).reshape(1, 1, -1)
    xx = pos[..., None].astype(np.float32) * np.float32(np.pi) * 2.0 * freqs
    enc = np.concatenate([np.sin(xx), np.cos(xx)], -1).reshape(N, -1)
    posf = np.concatenate([pos, enc], 1).astype(np.float32)
    masks = np.stack([(x > 0).astype(np.float32),
                      (x < n - 1).astype(np.float32)], 1)
    return morton_of_grid, grid_of_morton, np.concatenate([posf, masks], 1)


_TABS = [_tables(d) for d in range(MAXD + 1)]
_POS7_M = _TABS[7][2][:, :39][_TABS[7][1]]

_F_DEPTHS = list(range(2, 7))
_F_OFF = {}
_o = 0
for _d in _F_DEPTHS:
    _F_OFF[_d] = _o
    _o += ND[_d]
_F_TOT = _o
_F_CHUNK = 176
_F_PAD = _F_CHUNK * NWORK
_F_SRC = np.zeros(_F_PAD, np.int32)
for _d in _F_DEPTHS:
    _F_SRC[_F_OFF[_d]:_F_OFF[_d] + ND[_d]] = _F_OFF[_d] + _TABS[_d][0]

_SX = {}
for _d in range(1, MAXD):
    _n = 1 << _d
    _m = _n // 2
    _s = np.zeros((max(_m, 8), _n), np.float32)
    for _X in range(_m):
        _s[_X, 2 * _X] = 0.25
        _s[_X, 2 * _X + 1] = 0.25
    _SX[_d] = _s

_S4 = np.zeros((128, 512), np.float32)
for _r in range(128):
    _S4[_r, 4 * _r:4 * _r + 4] = 0.25


@functools.lru_cache(maxsize=None)
def _sc_mesh():
    return plsc.VectorSubcoreMesh(core_axis_name="c", subcore_axis_name="s",
                                  num_cores=2, num_subcores=16)


def _sc_mid_body(src_hbm, idx_hbm, p_hbm, pidx_hbm, out_hbm, pout_hbm,
                 tab_v, idx_v, out_v, pidx_v, prow_v, sem):
    w = lax.axis_index("s") * 2 + lax.axis_index("c")
    base = w * _F_CHUNK
    pbase = w * 128
    pltpu.sync_copy(pidx_hbm.at[pl.ds(pbase, 128)], pidx_v)
    pdesc = pltpu.async_copy(p_hbm.at[pidx_v], prow_v, sem)
    pltpu.sync_copy(src_hbm, tab_v)
    pltpu.sync_copy(idx_hbm.at[pl.ds(base, _F_CHUNK)], idx_v)

    def body(i, c):
        ids = idx_v[pl.ds(i * 16, 16)]
        out_v[pl.ds(i * 16, 16)] = plsc.load_gather(tab_v, [ids])
        return c

    lax.fori_loop(0, _F_CHUNK // 16, body, 0)
    pltpu.sync_copy(out_v, out_hbm.at[pl.ds(base, _F_CHUNK)])
    pdesc.wait()
    pltpu.sync_copy(prow_v, pout_hbm.at[pl.ds(pbase, 128)])


@functools.lru_cache(maxsize=None)
def _sc_mid():
    return pl.kernel(
        _sc_mid_body,
        out_type=(jax.ShapeDtypeStruct((_F_PAD,), jnp.float32),
                  jax.ShapeDtypeStruct((ND[6], H), jnp.float32)),
        mesh=_sc_mesh(),
        compiler_params=pltpu.CompilerParams(needs_layout_passes=False),
        scratch_types=[
            pltpu.VMEM((_F_PAD,), jnp.float32),
            pltpu.VMEM((_F_CHUNK,), jnp.int32),
            pltpu.VMEM((_F_CHUNK,), jnp.float32),
            pltpu.VMEM((128,), jnp.int32),
            pltpu.VMEM((128, H), jnp.float32),
            pltpu.SemaphoreType.DMA,
        ],
    )


_E_DEPTHS = [6, 5, 4, 3, 2]
_E_ALL = {6: 128, 5: 32, 4: 8}
_E_W0 = {3: 64, 2: 16}
_E_BOFF = {6: 0, 5: 128, 4: 160, 3: 168, 2: 232}
_E_BUF = 248


def _sc_unshuffle_body(*refs):
    ne = len(_E_DEPTHS)
    e_refs = dict(zip(_E_DEPTHS, refs[0:ne]))
    i_refs = dict(zip(_E_DEPTHS, refs[ne:2 * ne]))
    o_refs = dict(zip(_E_DEPTHS, refs[2 * ne:3 * ne]))
    idx_v, rows_v, sem = refs[3 * ne:]
    w = lax.axis_index("s") * 2 + lax.axis_index("c")

    for d, rw in _E_ALL.items():
        boff = _E_BOFF[d]
        pltpu.sync_copy(i_refs[d].at[pl.ds(w * rw, rw)],
                        idx_v.at[pl.ds(boff, rw)])
    descs = []
    for d, rw in _E_ALL.items():
        boff = _E_BOFF[d]
        descs.append(pltpu.async_copy(
            e_refs[d].at[idx_v.at[pl.ds(boff, rw)]],
            rows_v.at[pl.ds(boff, rw)], sem))
    for de in descs:
        de.wait()
    for d, rw in _E_ALL.items():
        boff = _E_BOFF[d]
        pltpu.sync_copy(rows_v.at[pl.ds(boff, rw)],
                        o_refs[d].at[pl.ds(w * rw, rw)])

    @pl.when(w == 0)
    def _():
        for d, rw in _E_W0.items():
            boff = _E_BOFF[d]
            pltpu.sync_copy(i_refs[d], idx_v.at[pl.ds(boff, rw)])
        descs0 = []
        for d, rw in _E_W0.items():
            boff = _E_BOFF[d]
            descs0.append(pltpu.async_copy(
                e_refs[d].at[idx_v.at[pl.ds(boff, rw)]],
                rows_v.at[pl.ds(boff, rw)], sem))
        for de in descs0:
            de.wait()
        for d, rw in _E_W0.items():
            boff = _E_BOFF[d]
            pltpu.sync_copy(rows_v.at[pl.ds(boff, rw)], o_refs[d])


@functools.lru_cache(maxsize=None)
def _sc_unshuffle():
    return pl.kernel(
        _sc_unshuffle_body,
        out_type=tuple(jax.ShapeDtypeStruct((ND[d], H), jnp.float32)
                       for d in _E_DEPTHS),
        mesh=_sc_mesh(),
        scratch_types=[
            pltpu.VMEM((_E_BUF,), jnp.int32),
            pltpu.VMEM((_E_BUF, H), jnp.float32),
            pltpu.SemaphoreType.DMA,
        ],
    )


def _shift_zero(T, s, N):
    if s == 0:
        return T
    z = jnp.zeros((abs(s), T.shape[1]), T.dtype)
    if s > 0:
        return jnp.concatenate([T[s:], z], axis=0)
    return jnp.concatenate([z, T[:s]], axis=0)


def _layernorm(z, g2, b2):
    mu = jnp.mean(z, axis=1, keepdims=True)
    zc = z - mu
    var = jnp.mean(zc * zc, axis=1, keepdims=True)
    return zc * lax.rsqrt(var + 1e-5) * g2 + b2


def _tca_body(X7, inW, inb, embW7, embb7, g27, b27, S4, E7, P6):
    h = jnp.dot(X7[...].astype(BF16), inW[...],
                preferred_element_type=jnp.float32) + inb[...]
    E7[...] = h
    for c in range(32):
        blk = E7[pl.ds(c * 512, 512), :].astype(BF16)
        P6[pl.ds(c * 128, 128), :] = jnp.dot(S4[...], blk,
                                             preferred_element_type=jnp.float32)
    z = jnp.dot(h.astype(BF16), embW7[...],
                preferred_element_type=jnp.float32) + embb7[...]
    E7[...] = _layernorm(z, g27[...], b27[...])


_tca = pl.pallas_call(
    _tca_body,
    out_shape=(jax.ShapeDtypeStruct((ND[7], H), jnp.float32),
               jax.ShapeDtypeStruct((ND[6], H), jnp.float32)),
)


def _tcb_body(*refs):
    it = iter(refs)
    X = [next(it) for _ in range(7)]
    P6g = next(it)
    inW = next(it)
    inb = next(it)
    convW = next(it)
    convb = next(it)
    embW = next(it)
    embb = next(it)
    g2 = next(it)
    b2 = next(it)
    Sx = {d: next(it) for d in range(1, 7)}
    E = [next(it) for _ in range(7)]

    Wv = inW[...]
    bv = inb[...]
    for d in range(7):
        A = X[d][...][:, 0:40].astype(BF16)
        h0 = jnp.dot(A, Wv, preferred_element_type=jnp.float32) + bv
        if d == 6:
            h0 = h0 + P6g[...]
        E[d][...] = h0

    def conv(dc):
        nc = 1 << dc
        Nc = ND[dc]
        hv = E[dc][...].astype(BF16)
        Xm = X[dc][:, 40:42]
        sums = []
        for dx in (-1, 0, 1):
            t = None
            for dy in (-1, 0, 1):
                j = (dy + 1) * 3 + (dx + 1)
                T = jnp.dot(hv, convW[dc - 1, j * H:(j + 1) * H, :],
                            preferred_element_type=jnp.float32)
                T = _shift_zero(T, dy * nc + dx, Nc)
                t = T if t is None else t + T
            sums.append(t)
        acc = (convb[dc - 1:dc, :] + sums[1]
               + Xm[:, 0:1] * sums[0] + Xm[:, 1:2] * sums[2])
        E[dc][...] = jnp.maximum(acc, 0.0)

    conv(6)
    for d in range(6, 0, -1):
        n = 1 << d
        m = n // 2
        Sxv = Sx[d][...]
        for Y in range(m):
            rA = E[d][pl.ds((2 * Y) * n, n), :]
            rB = E[d][pl.ds((2 * Y + 1) * n, n), :]
            ch = jnp.dot(Sxv, (rA + rB).astype(BF16),
                         preferred_element_type=jnp.float32)
            E[d - 1][pl.ds(Y * m, m), :] = E[d - 1][pl.ds(Y * m, m), :] + ch[:m]
        if d - 1 >= 1:
            conv(d - 1)

    for d in range(7):
        hv = E[d][...].astype(BF16)
        z = jnp.dot(hv, embW[d], preferred_element_type=jnp.float32) + embb[d:d + 1, :]
        E[d][...] = _layernorm(z, g2[d:d + 1, :], b2[d:d + 1, :])


_tcb = pl.pallas_call(
    _tcb_body,
    out_shape=tuple(jax.ShapeDtypeStruct((ND[d], H), jnp.float32)
                    for d in range(7)),
)


def kernel(features_0, features_1, features_2, features_3, features_4,
           features_5, features_6, features_7, in_proj_W, in_proj_b,
           conv_W, conv_b, emb_W, emb_b, ln_g, ln_b, depth_gain):
    feats = [features_0, features_1, features_2, features_3, features_4,
             features_5, features_6, features_7]
    f32 = jnp.float32

    inW16 = in_proj_W.astype(BF16)
    inb2 = in_proj_b.reshape(1, H)
    embW16 = emb_W.astype(BF16)
    g2 = depth_gain[:, None] * ln_g
    b2 = depth_gain[:, None] * ln_b

    X7 = jnp.concatenate([feats[7], jnp.asarray(_POS7_M)], axis=1)
    E7, P6m = _tca(X7, inW16, inb2, embW16[7], emb_b[7].reshape(1, H),
                   g2[7].reshape(1, H), b2[7].reshape(1, H),
                   jnp.asarray(_S4).astype(BF16))

    fall = jnp.concatenate(
        [feats[d].reshape(-1) for d in _F_DEPTHS]
        + [jnp.zeros((_F_PAD - _F_TOT,), f32)])
    fg, P6g = _sc_mid()(fall, jnp.asarray(_F_SRC),
                        P6m, jnp.asarray(_TABS[6][0]))

    ops = []
    for d in range(7):
        if d < 2:
            fcol = feats[d]
        else:
            fcol = fg[_F_OFF[d]:_F_OFF[d] + ND[d]].reshape(ND[d], 1)
        ops.append(jnp.concatenate([fcol, jnp.asarray(_TABS[d][2])], axis=1))
    ops.append(P6g)
    ops.append(inW16)
    ops.append(inb2)
    ops.append(conv_W[1:7].astype(BF16))
    ops.append(conv_b[1:7])
    ops.append(embW16[:7])
    ops.append(emb_b[:7])
    ops.append(g2[:7])
    ops.append(b2[:7])
    for d in range(1, 7):
        ops.append(jnp.asarray(_SX[d]).astype(BF16))
    Eg = _tcb(*ops)

    Em = _sc_unshuffle()(
        *[Eg[d] for d in _E_DEPTHS],
        *[jnp.asarray(_TABS[d][1]) for d in _E_DEPTHS])
    EmD = dict(zip(_E_DEPTHS, Em))

    return (Eg[0], Eg[1], EmD[2], EmD[3], EmD[4], EmD[5], EmD[6], E7)
